# baseline (device time: 9903 ns/iter reference)
import jax
import jax.numpy as jnp
from jax import lax
from jax.experimental import pallas as pl
from jax.experimental.pallas import tpu as pltpu

N_DEV = 8


def kernel(x):
    m, n = x.shape

    def body(
        x_ref,
        out_ref,
        send_buf,
        gather_ref,
        credit_src,
        credit_dst,
        send_sems,
        recv_sems,
        csend_sems,
        crecv_sems,
    ):
        my = lax.axis_index("i")

        bar = pltpu.get_barrier_semaphore()
        pl.semaphore_signal(
            bar, inc=1, device_id=(my,), device_id_type=pl.DeviceIdType.MESH
        )
        pl.semaphore_wait(bar, 1)

        send_buf[0:1, :] = jnp.sum(x_ref[...], axis=0, keepdims=True)

        for off in range(1, N_DEV):

            @pl.when(my + off < N_DEV)
            def _():
                rdma = pltpu.make_async_remote_copy(
                    src_ref=send_buf.at[pl.ds(0, 1)],
                    dst_ref=gather_ref.at[my],
                    send_sem=send_sems.at[off - 1],
                    recv_sem=recv_sems.at[my],
                    device_id=(my + off,),
                    device_id_type=pl.DeviceIdType.MESH,
                )
                rdma.start()

        r = lax.broadcasted_iota(jnp.int32, (m, m), 0)
        c = lax.broadcasted_iota(jnp.int32, (m, m), 1)
        tril = (r >= c).astype(jnp.float32)
        out_ref[...] = lax.dot_general(
            tril,
            x_ref[...],
            dimension_numbers=(((1,), (0,)), ((), ())),
            preferred_element_type=jnp.float32,
        )

        for j in range(N_DEV - 1):

            @pl.when(j < my)
            def _():
                recv = pltpu.make_async_remote_copy(
                    src_ref=send_buf.at[pl.ds(0, 1)],
                    dst_ref=gather_ref.at[j],
                    send_sem=send_sems.at[N_DEV - 1],
                    recv_sem=recv_sems.at[j],
                    device_id=(0,),
                    device_id_type=pl.DeviceIdType.MESH,
                )
                recv.wait_recv()

        vals = gather_ref[:, 0, :]
        row = lax.broadcasted_iota(jnp.int32, (N_DEV, n), 0)
        offset = jnp.sum(
            jnp.where(row < my, vals, 0.0), axis=0, keepdims=True
        )
        out_ref[...] = out_ref[...] + offset

        for j in range(N_DEV - 1):

            @pl.when(j < my)
            def _():
                credit = pltpu.make_async_remote_copy(
                    src_ref=credit_src,
                    dst_ref=credit_dst.at[my],
                    send_sem=csend_sems.at[j],
                    recv_sem=crecv_sems.at[my],
                    device_id=(j,),
                    device_id_type=pl.DeviceIdType.MESH,
                )
                credit.start()

        for off in range(1, N_DEV):

            @pl.when(my + off < N_DEV)
            def _():
                send = pltpu.make_async_remote_copy(
                    src_ref=send_buf.at[pl.ds(0, 1)],
                    dst_ref=gather_ref.at[0],
                    send_sem=send_sems.at[off - 1],
                    recv_sem=recv_sems.at[N_DEV - 1],
                    device_id=(0,),
                    device_id_type=pl.DeviceIdType.MESH,
                )
                send.wait_send()

        for off in range(1, N_DEV):

            @pl.when(my + off < N_DEV)
            def _():
                credit = pltpu.make_async_remote_copy(
                    src_ref=credit_src,
                    dst_ref=credit_dst.at[my + off],
                    send_sem=csend_sems.at[N_DEV - 1],
                    recv_sem=crecv_sems.at[my + off],
                    device_id=(0,),
                    device_id_type=pl.DeviceIdType.MESH,
                )
                credit.wait_recv()

        for j in range(N_DEV - 1):

            @pl.when(j < my)
            def _():
                credit = pltpu.make_async_remote_copy(
                    src_ref=credit_src,
                    dst_ref=credit_dst.at[0],
                    send_sem=csend_sems.at[j],
                    recv_sem=crecv_sems.at[N_DEV - 1],
                    device_id=(0,),
                    device_id_type=pl.DeviceIdType.MESH,
                )
                credit.wait_send()

    return pl.pallas_call(
        body,
        out_shape=jax.ShapeDtypeStruct((m, n), jnp.float32),
        in_specs=[pl.BlockSpec(memory_space=pltpu.VMEM)],
        out_specs=pl.BlockSpec(memory_space=pltpu.VMEM),
        scratch_shapes=[
            pltpu.VMEM((8, n), jnp.float32),
            pltpu.VMEM((N_DEV, 1, n), jnp.float32),
            pltpu.VMEM((1, 128), jnp.float32),
            pltpu.VMEM((N_DEV, 1, 128), jnp.float32),
            pltpu.SemaphoreType.DMA((N_DEV,)),
            pltpu.SemaphoreType.DMA((N_DEV,)),
            pltpu.SemaphoreType.DMA((N_DEV,)),
            pltpu.SemaphoreType.DMA((N_DEV,)),
        ],
        compiler_params=pltpu.CompilerParams(collective_id=0),
    )(x)


# device time: 9005 ns/iter; 1.0997x vs baseline; 1.0997x over previous
import jax
import jax.numpy as jnp
from jax import lax
from jax.experimental import pallas as pl
from jax.experimental.pallas import tpu as pltpu

N_DEV = 8


def kernel(x):
    m, n = x.shape

    def body(
        x_ref,
        out_ref,
        send_buf,
        gather_ref,
        credit_src,
        credit_dst,
        send_sems,
        recv_sems,
        csend_sems,
        crecv_sems,
    ):
        my = lax.axis_index("i")

        bar = pltpu.get_barrier_semaphore()
        pl.semaphore_signal(
            bar, inc=1, device_id=(my,), device_id_type=pl.DeviceIdType.MESH
        )
        pl.semaphore_wait(bar, 1)

        send_buf[0:1, :] = jnp.sum(x_ref[...], axis=0, keepdims=True)

        for off in range(1, N_DEV):

            @pl.when(my + off < N_DEV)
            def _():
                rdma = pltpu.make_async_remote_copy(
                    src_ref=send_buf.at[pl.ds(0, 1)],
                    dst_ref=gather_ref.at[my],
                    send_sem=send_sems.at[off - 1],
                    recv_sem=recv_sems.at[my],
                    device_id=(my + off,),
                    device_id_type=pl.DeviceIdType.MESH,
                )
                rdma.start()

        r = lax.broadcasted_iota(jnp.int32, (m, m), 0)
        c = lax.broadcasted_iota(jnp.int32, (m, m), 1)
        tril = (r >= c).astype(jnp.float32)
        out_ref[...] = lax.dot_general(
            tril,
            x_ref[...],
            dimension_numbers=(((1,), (0,)), ((), ())),
            preferred_element_type=jnp.float32,
        )

        acc = jnp.zeros((1, n), jnp.float32)
        for j in range(N_DEV - 1):

            @pl.when(j < my)
            def _():
                recv = pltpu.make_async_remote_copy(
                    src_ref=send_buf.at[pl.ds(0, 1)],
                    dst_ref=gather_ref.at[j],
                    send_sem=send_sems.at[N_DEV - 1],
                    recv_sem=recv_sems.at[j],
                    device_id=(0,),
                    device_id_type=pl.DeviceIdType.MESH,
                )
                recv.wait_recv()

            acc = acc + jnp.where(j < my, gather_ref[j], 0.0)

            @pl.when(j < my)
            def _():
                credit = pltpu.make_async_remote_copy(
                    src_ref=credit_src,
                    dst_ref=credit_dst.at[my],
                    send_sem=csend_sems.at[j],
                    recv_sem=crecv_sems.at[my],
                    device_id=(j,),
                    device_id_type=pl.DeviceIdType.MESH,
                )
                credit.start()

        out_ref[...] = out_ref[...] + acc

        for off in range(1, N_DEV):

            @pl.when(my + off < N_DEV)
            def _():
                send = pltpu.make_async_remote_copy(
                    src_ref=send_buf.at[pl.ds(0, 1)],
                    dst_ref=gather_ref.at[0],
                    send_sem=send_sems.at[off - 1],
                    recv_sem=recv_sems.at[N_DEV - 1],
                    device_id=(0,),
                    device_id_type=pl.DeviceIdType.MESH,
                )
                send.wait_send()

        for off in range(1, N_DEV):

            @pl.when(my + off < N_DEV)
            def _():
                credit = pltpu.make_async_remote_copy(
                    src_ref=credit_src,
                    dst_ref=credit_dst.at[my + off],
                    send_sem=csend_sems.at[N_DEV - 1],
                    recv_sem=crecv_sems.at[my + off],
                    device_id=(0,),
                    device_id_type=pl.DeviceIdType.MESH,
                )
                credit.wait_recv()

        for j in range(N_DEV - 1):

            @pl.when(j < my)
            def _():
                credit = pltpu.make_async_remote_copy(
                    src_ref=credit_src,
                    dst_ref=credit_dst.at[0],
                    send_sem=csend_sems.at[j],
                    recv_sem=crecv_sems.at[N_DEV - 1],
                    device_id=(0,),
                    device_id_type=pl.DeviceIdType.MESH,
                )
                credit.wait_send()

    return pl.pallas_call(
        body,
        out_shape=jax.ShapeDtypeStruct((m, n), jnp.float32),
        in_specs=[pl.BlockSpec(memory_space=pltpu.VMEM)],
        out_specs=pl.BlockSpec(memory_space=pltpu.VMEM),
        scratch_shapes=[
            pltpu.VMEM((8, n), jnp.float32),
            pltpu.VMEM((N_DEV, 1, n), jnp.float32),
            pltpu.VMEM((1, 128), jnp.float32),
            pltpu.VMEM((N_DEV, 1, 128), jnp.float32),
            pltpu.SemaphoreType.DMA((N_DEV,)),
            pltpu.SemaphoreType.DMA((N_DEV,)),
            pltpu.SemaphoreType.DMA((N_DEV,)),
            pltpu.SemaphoreType.DMA((N_DEV,)),
        ],
        compiler_params=pltpu.CompilerParams(collective_id=0),
    )(x)


# device time: 8050 ns/iter; 1.2302x vs baseline; 1.1186x over previous
import jax
import jax.numpy as jnp
from jax import lax
from jax.experimental import pallas as pl
from jax.experimental.pallas import tpu as pltpu

N_DEV = 8


def kernel(x):
    m, n = x.shape

    def body(x_ref, out_ref, send_buf, gather_ref, send_sems, recv_sems):
        my = lax.axis_index("i")

        bar = pltpu.get_barrier_semaphore()
        for j in range(N_DEV - 1):

            @pl.when(j < my)
            def _():
                pl.semaphore_signal(
                    bar,
                    inc=1,
                    device_id=(j,),
                    device_id_type=pl.DeviceIdType.MESH,
                )

        send_buf[0:1, :] = jnp.sum(x_ref[...], axis=0, keepdims=True)

        for off in range(1, N_DEV):

            @pl.when(my + off < N_DEV)
            def _():
                pl.semaphore_wait(bar, 1)

        for off in range(1, N_DEV):

            @pl.when(my + off < N_DEV)
            def _():
                rdma = pltpu.make_async_remote_copy(
                    src_ref=send_buf.at[pl.ds(0, 1)],
                    dst_ref=gather_ref.at[my],
                    send_sem=send_sems.at[off - 1],
                    recv_sem=recv_sems.at[my],
                    device_id=(my + off,),
                    device_id_type=pl.DeviceIdType.MESH,
                )
                rdma.start()

        r = lax.broadcasted_iota(jnp.int32, (m, m), 0)
        c = lax.broadcasted_iota(jnp.int32, (m, m), 1)
        tril = (r >= c).astype(jnp.float32)
        out_ref[...] = lax.dot_general(
            tril,
            x_ref[...],
            dimension_numbers=(((1,), (0,)), ((), ())),
            preferred_element_type=jnp.float32,
        )

        for j in range(N_DEV - 1):

            @pl.when(j < my)
            def _():
                recv = pltpu.make_async_remote_copy(
                    src_ref=send_buf.at[pl.ds(0, 1)],
                    dst_ref=gather_ref.at[j],
                    send_sem=send_sems.at[N_DEV - 1],
                    recv_sem=recv_sems.at[j],
                    device_id=(0,),
                    device_id_type=pl.DeviceIdType.MESH,
                )
                recv.wait_recv()

        vals = gather_ref[:, 0, :]
        row = lax.broadcasted_iota(jnp.int32, (N_DEV, n), 0)
        offset = jnp.sum(
            jnp.where(row < my, vals, 0.0), axis=0, keepdims=True
        )
        out_ref[...] = out_ref[...] + offset

        for off in range(1, N_DEV):

            @pl.when(my + off < N_DEV)
            def _():
                send = pltpu.make_async_remote_copy(
                    src_ref=send_buf.at[pl.ds(0, 1)],
                    dst_ref=gather_ref.at[0],
                    send_sem=send_sems.at[off - 1],
                    recv_sem=recv_sems.at[N_DEV - 1],
                    device_id=(0,),
                    device_id_type=pl.DeviceIdType.MESH,
                )
                send.wait_send()

    return pl.pallas_call(
        body,
        out_shape=jax.ShapeDtypeStruct((m, n), jnp.float32),
        in_specs=[pl.BlockSpec(memory_space=pltpu.VMEM)],
        out_specs=pl.BlockSpec(memory_space=pltpu.VMEM),
        scratch_shapes=[
            pltpu.VMEM((8, n), jnp.float32),
            pltpu.VMEM((N_DEV, 1, n), jnp.float32),
            pltpu.SemaphoreType.DMA((N_DEV,)),
            pltpu.SemaphoreType.DMA((N_DEV,)),
        ],
        compiler_params=pltpu.CompilerParams(collective_id=0),
    )(x)
